# baseline (speedup 1.0000x reference)
import functools

import jax
import jax.numpy as jnp
from jax import lax
from jax.experimental import pallas as pl
from jax.experimental.pallas import tpu as pltpu
from jax.experimental.pallas import tpu_sc as plsc

VOCAB = 1000000
EMBED = 300
BATCH = 1024
SEQ = 512
HIDDEN = 16

NC = 2
NS = 16
NW = NC * NS
ROWS_PER_W = BATCH // NW
CHUNK = 128
CHUNKS_PER_ROW = SEQ // CHUNK
CHUNKS_PER_W = ROWS_PER_W * CHUNKS_PER_ROW

TAIL = 256
TAIL_W = EMBED - TAIL
SLOTS = 17
SUM_W = 16 * SLOTS
SLOT_SRC = tuple((j // 8, 16 * (j % 8)) for j in range(16)) + ((2, 0),)

P2_BLOCK = 5000


def _sc_pool_body(tok_hbm, table_hbm, p2_hbm, out_hbm, tok_v, idx_v, buf0,
                  buf1, buf2, acc_v, sems):
    wid = lax.axis_index("s") * NC + lax.axis_index("c")
    bufs = (buf0, buf1, buf2)
    pltpu.sync_copy(tok_hbm.at[pl.ds(wid * CHUNKS_PER_W, CHUNKS_PER_W)], tok_v)

    def zero_row(i, carry):
        for j in range(SLOTS):
            acc_v[i, pl.ds(16 * j, 16)] = jnp.zeros((16,), jnp.float32)
        return carry
    lax.fori_loop(0, ROWS_PER_W, zero_row, 0)

    def copy_idx(c, parity):
        for v in range(8):
            idx_v[parity, pl.ds(16 * v, 16)] = tok_v[c, pl.ds(16 * v, 16)]

    def srcs(parity):
        return (
            table_hbm.at[idx_v.at[parity], pl.ds(0, 128)],
            table_hbm.at[idx_v.at[parity], pl.ds(128, 128)],
            p2_hbm.at[idx_v.at[parity]],
        )

    def start_gathers(parity):
        for g, src in enumerate(srcs(parity)):
            pltpu.make_async_copy(src, bufs[g].at[parity],
                                  sems.at[parity, g]).start()

    def wait_gathers(parity):
        for g, src in enumerate(srcs(parity)):
            pltpu.make_async_copy(src, bufs[g].at[parity],
                                  sems.at[parity, g]).wait()

    copy_idx(0, 0)
    start_gathers(0)

    def chunk_body(c, parity):
        @pl.when(c < CHUNKS_PER_W - 1)
        def _():
            copy_idx(c + 1, 1 - parity)
            start_gathers(1 - parity)
        wait_gathers(parity)
        racc = c // CHUNKS_PER_ROW

        accs0 = tuple(acc_v[racc, pl.ds(16 * j, 16)] for j in range(SLOTS))

        def accum4(r, accs):
            accs = list(accs)
            for rr in range(4):
                row = 4 * r + rr
                for j, (g, off) in enumerate(SLOT_SRC):
                    accs[j] = accs[j] + bufs[g][parity, row, pl.ds(off, 16)]
            return tuple(accs)

        accs = lax.fori_loop(0, CHUNK // 4, accum4, accs0)
        for j in range(SLOTS):
            acc_v[racc, pl.ds(16 * j, 16)] = accs[j]

    def pair_body(g, carry):
        chunk_body(2 * g, 0)
        chunk_body(2 * g + 1, 1)
        return carry

    lax.fori_loop(0, CHUNKS_PER_W // 2, pair_body, 0)

    pltpu.sync_copy(acc_v, out_hbm.at[pl.ds(wid * ROWS_PER_W, ROWS_PER_W)])


_sc_pool = functools.partial(
    pl.kernel,
    mesh=plsc.VectorSubcoreMesh(core_axis_name="c", subcore_axis_name="s"),
    out_type=jax.ShapeDtypeStruct((BATCH, SUM_W), jnp.float32),
    scratch_types=[
        pltpu.VMEM((CHUNKS_PER_W, CHUNK), jnp.int32),
        pltpu.VMEM((2, CHUNK), jnp.int32),
        pltpu.VMEM((2, CHUNK, 128), jnp.float32),
        pltpu.VMEM((2, CHUNK, 128), jnp.float32),
        pltpu.VMEM((2, CHUNK, 128), jnp.float32),
        pltpu.VMEM((ROWS_PER_W, SUM_W), jnp.float32),
        pltpu.SemaphoreType.DMA((2, 3)),
    ],
)(_sc_pool_body)


def _p2_body(x_ref, wt_ref, o_ref):
    x = x_ref[:, TAIL:EMBED]
    h = jnp.dot(x, wt_ref[...], preferred_element_type=jnp.float32)
    o_ref[...] = jnp.zeros_like(o_ref)
    o_ref[:, 0:HIDDEN] = h


def _mlp_body(x_ref, w1_ref, b1_ref, w2_ref, b2_ref, o_ref):
    x = x_ref[...]
    h = jnp.dot(x, w1_ref[...], preferred_element_type=jnp.float32)
    h = jnp.maximum(h + b1_ref[...], 0.0)
    o = jnp.dot(h, w2_ref[...], preferred_element_type=jnp.float32)
    o_ref[...] = jax.nn.sigmoid(o + b2_ref[...])


def kernel(tokens, emb_table, W1, b1, W2, b2):
    tok = tokens.reshape(BATCH * CHUNKS_PER_ROW, CHUNK)

    p2 = pl.pallas_call(
        _p2_body,
        grid=(VOCAB // P2_BLOCK,),
        in_specs=[
            pl.BlockSpec((P2_BLOCK, EMBED), lambda i: (i, 0)),
            pl.BlockSpec((TAIL_W, HIDDEN), lambda i: (0, 0)),
        ],
        out_specs=pl.BlockSpec((P2_BLOCK, 128), lambda i: (i, 0)),
        out_shape=jax.ShapeDtypeStruct((VOCAB, 128), jnp.float32),
    )(emb_table, W1[TAIL:EMBED])

    sums = _sc_pool(tok, emb_table, p2)

    w1_ext = jnp.concatenate(
        [W1[:TAIL], jnp.eye(HIDDEN, dtype=W1.dtype)], axis=0) * (1.0 / SEQ)

    out = pl.pallas_call(
        _mlp_body,
        out_shape=jax.ShapeDtypeStruct((BATCH, 1), jnp.float32),
    )(sums, w1_ext, b1.reshape(1, HIDDEN), W2, b2.reshape(1, 1))
    return out


# speedup vs baseline: 1.1568x; 1.1568x over previous
import functools

import jax
import jax.numpy as jnp
from jax import lax
from jax.experimental import pallas as pl
from jax.experimental.pallas import tpu as pltpu
from jax.experimental.pallas import tpu_sc as plsc

VOCAB = 1000000
EMBED = 300
BATCH = 1024
SEQ = 512
HIDDEN = 16

NC = 2
NS = 16
NW = NC * NS
ROWS_PER_W = BATCH // NW
CHUNK = 128
CHUNKS_PER_ROW = SEQ // CHUNK
CHUNKS_PER_W = ROWS_PER_W * CHUNKS_PER_ROW

TAIL = 256
TAIL_W = EMBED - TAIL
SLOTS = 17
SUM_W = 16 * SLOTS
SLOT_SRC = tuple((j // 8, 16 * (j % 8)) for j in range(16)) + ((2, 0),)

P2_BLOCK = 20000


def _sc_pool_body(tok_hbm, table_hbm, p2_hbm, out_hbm, tok_v, idx_v, buf0,
                  buf1, buf2, acc_v, sems):
    wid = lax.axis_index("s") * NC + lax.axis_index("c")
    bufs = (buf0, buf1, buf2)
    pltpu.sync_copy(tok_hbm.at[pl.ds(wid * CHUNKS_PER_W, CHUNKS_PER_W)], tok_v)

    def zero_row(i, carry):
        for j in range(SLOTS):
            acc_v[i, pl.ds(16 * j, 16)] = jnp.zeros((16,), jnp.float32)
        return carry
    lax.fori_loop(0, ROWS_PER_W, zero_row, 0)

    def copy_idx(c, parity):
        for v in range(8):
            idx_v[parity, pl.ds(16 * v, 16)] = tok_v[c, pl.ds(16 * v, 16)]

    def srcs(parity):
        return (
            table_hbm.at[idx_v.at[parity], pl.ds(0, 128)],
            table_hbm.at[idx_v.at[parity], pl.ds(128, 128)],
            p2_hbm.at[idx_v.at[parity]],
        )

    def start_gathers(parity):
        for g, src in enumerate(srcs(parity)):
            pltpu.make_async_copy(src, bufs[g].at[parity],
                                  sems.at[parity, g]).start()

    def wait_gathers(parity):
        for g, src in enumerate(srcs(parity)):
            pltpu.make_async_copy(src, bufs[g].at[parity],
                                  sems.at[parity, g]).wait()

    copy_idx(0, 0)
    start_gathers(0)

    def chunk_body(c, parity):
        @pl.when(c < CHUNKS_PER_W - 1)
        def _():
            copy_idx(c + 1, 1 - parity)
            start_gathers(1 - parity)
        wait_gathers(parity)
        racc = c // CHUNKS_PER_ROW

        accs0 = tuple(acc_v[racc, pl.ds(16 * j, 16)] for j in range(SLOTS))

        def accum4(r, accs):
            accs = list(accs)
            for rr in range(4):
                row = 4 * r + rr
                for j, (g, off) in enumerate(SLOT_SRC):
                    accs[j] = accs[j] + bufs[g][parity, row, pl.ds(off, 16)]
            return tuple(accs)

        accs = lax.fori_loop(0, CHUNK // 4, accum4, accs0)
        for j in range(SLOTS):
            acc_v[racc, pl.ds(16 * j, 16)] = accs[j]

    def pair_body(g, carry):
        chunk_body(2 * g, 0)
        chunk_body(2 * g + 1, 1)
        return carry

    lax.fori_loop(0, CHUNKS_PER_W // 2, pair_body, 0)

    pltpu.sync_copy(acc_v, out_hbm.at[pl.ds(wid * ROWS_PER_W, ROWS_PER_W)])


_sc_pool = functools.partial(
    pl.kernel,
    mesh=plsc.VectorSubcoreMesh(core_axis_name="c", subcore_axis_name="s"),
    out_type=jax.ShapeDtypeStruct((BATCH, SUM_W), jnp.float32),
    scratch_types=[
        pltpu.VMEM((CHUNKS_PER_W, CHUNK), jnp.int32),
        pltpu.VMEM((2, CHUNK), jnp.int32),
        pltpu.VMEM((2, CHUNK, 128), jnp.float32),
        pltpu.VMEM((2, CHUNK, 128), jnp.float32),
        pltpu.VMEM((2, CHUNK, 128), jnp.float32),
        pltpu.VMEM((ROWS_PER_W, SUM_W), jnp.float32),
        pltpu.SemaphoreType.DMA((2, 3)),
    ],
)(_sc_pool_body)


def _p2_body(x_ref, wt_ref, o_ref):
    lane = lax.broadcasted_iota(jnp.int32, x_ref.shape, 1)
    x = jnp.where(lane < TAIL_W, x_ref[...], 0.0)
    h = jnp.dot(x, wt_ref[...], preferred_element_type=jnp.float32)
    o_ref[...] = jnp.zeros_like(o_ref)
    o_ref[:, 0:HIDDEN] = h


def _mlp_body(x_ref, w1_ref, b1_ref, w2_ref, b2_ref, o_ref):
    x = x_ref[...]
    h = jnp.dot(x, w1_ref[...], preferred_element_type=jnp.float32)
    h = jnp.maximum(h + b1_ref[...], 0.0)
    o = jnp.dot(h, w2_ref[...], preferred_element_type=jnp.float32)
    o_ref[...] = jax.nn.sigmoid(o + b2_ref[...])


def kernel(tokens, emb_table, W1, b1, W2, b2):
    tok = tokens.reshape(BATCH * CHUNKS_PER_ROW, CHUNK)

    wt = jnp.concatenate(
        [W1[TAIL:EMBED], jnp.zeros((128 - TAIL_W, HIDDEN), W1.dtype)], axis=0)
    p2 = pl.pallas_call(
        _p2_body,
        grid=(VOCAB // P2_BLOCK,),
        in_specs=[
            pl.BlockSpec((P2_BLOCK, 128), lambda i: (i, 2)),
            pl.BlockSpec((128, HIDDEN), lambda i: (0, 0)),
        ],
        out_specs=pl.BlockSpec((P2_BLOCK, 128), lambda i: (i, 0)),
        out_shape=jax.ShapeDtypeStruct((VOCAB, 128), jnp.float32),
    )(emb_table, wt)

    sums = _sc_pool(tok, emb_table, p2)

    w1_ext = jnp.concatenate(
        [W1[:TAIL], jnp.eye(HIDDEN, dtype=W1.dtype)], axis=0) * (1.0 / SEQ)

    out = pl.pallas_call(
        _mlp_body,
        out_shape=jax.ShapeDtypeStruct((BATCH, 1), jnp.float32),
    )(sums, w1_ext, b1.reshape(1, HIDDEN), W2, b2.reshape(1, 1))
    return out


# speedup vs baseline: 3.0583x; 2.6438x over previous
import functools

import jax
import jax.numpy as jnp
from jax import lax
from jax.experimental import pallas as pl
from jax.experimental.pallas import tpu as pltpu
from jax.experimental.pallas import tpu_sc as plsc

VOCAB = 1000000
EMBED = 300
BATCH = 1024
SEQ = 512
HIDDEN = 16

NC = 2
NS = 16
NW = NC * NS
ROWS_PER_W = BATCH // NW
CHUNK = 128
CHUNKS_PER_ROW = SEQ // CHUNK
CHUNKS_PER_W = ROWS_PER_W * CHUNKS_PER_ROW

P_BLOCK = 8192


def _sc_pool_body(tok_hbm, p_hbm, out_hbm, tok_v, idx_v, buf_v, acc_v, sems):
    wid = lax.axis_index("s") * NC + lax.axis_index("c")
    pltpu.sync_copy(tok_hbm.at[pl.ds(wid * CHUNKS_PER_W, CHUNKS_PER_W)], tok_v)

    def copy_idx(c, parity):
        for v in range(8):
            idx_v[parity, pl.ds(16 * v, 16)] = tok_v[c, pl.ds(16 * v, 16)]

    def start_gather(parity):
        pltpu.make_async_copy(p_hbm.at[idx_v.at[parity]], buf_v.at[parity],
                              sems.at[parity]).start()

    def wait_gather(parity):
        pltpu.make_async_copy(p_hbm.at[idx_v.at[parity]], buf_v.at[parity],
                              sems.at[parity]).wait()

    copy_idx(0, 0)
    start_gather(0)

    def chunk_body(c, parity):
        @pl.when(c < CHUNKS_PER_W - 1)
        def _():
            copy_idx(c + 1, 1 - parity)
            start_gather(1 - parity)
        wait_gather(parity)
        racc = c // CHUNKS_PER_ROW

        def accum8(r, acc):
            for rr in range(8):
                acc = acc + buf_v[parity, 8 * r + rr, pl.ds(0, 16)]
            return acc

        acc = lax.fori_loop(0, CHUNK // 8, accum8,
                            jnp.zeros((16,), jnp.float32))
        acc_v[racc, pl.ds(0, 16)] = acc_v[racc, pl.ds(0, 16)] + acc

    def pair_body(g, carry):
        chunk_body(2 * g, 0)
        chunk_body(2 * g + 1, 1)
        return carry

    def zero_row(i, carry):
        acc_v[i, pl.ds(0, 16)] = jnp.zeros((16,), jnp.float32)
        return carry
    lax.fori_loop(0, ROWS_PER_W, zero_row, 0)

    lax.fori_loop(0, CHUNKS_PER_W // 2, pair_body, 0)

    pltpu.sync_copy(acc_v, out_hbm.at[pl.ds(wid * ROWS_PER_W, ROWS_PER_W)])


_sc_pool = functools.partial(
    pl.kernel,
    mesh=plsc.VectorSubcoreMesh(core_axis_name="c", subcore_axis_name="s"),
    out_type=jax.ShapeDtypeStruct((BATCH, HIDDEN), jnp.float32),
    scratch_types=[
        pltpu.VMEM((CHUNKS_PER_W, CHUNK), jnp.int32),
        pltpu.VMEM((2, CHUNK), jnp.int32),
        pltpu.VMEM((2, CHUNK, 128), jnp.float32),
        pltpu.VMEM((ROWS_PER_W, HIDDEN), jnp.float32),
        pltpu.SemaphoreType.DMA((2,)),
    ],
)(_sc_pool_body)


def _proj_body(xt_ref, w_ref, o_ref):
    o_ref[...] = lax.dot_general(
        xt_ref[...], w_ref[...], (((0,), (0,)), ((), ())),
        preferred_element_type=jnp.float32)


def _mlp_body(x_ref, b1_ref, w2_ref, b2_ref, o_ref):
    h = jnp.maximum(x_ref[...] + b1_ref[...], 0.0)
    o = jnp.dot(h, w2_ref[...], preferred_element_type=jnp.float32)
    o_ref[...] = jax.nn.sigmoid(o + b2_ref[...])


def kernel(tokens, emb_table, W1, b1, W2, b2):
    tok = tokens.reshape(BATCH * CHUNKS_PER_ROW, CHUNK)

    embt = emb_table.T
    w1rep = jnp.tile(W1 * (1.0 / SEQ), (1, 128 // HIDDEN))
    p = pl.pallas_call(
        _proj_body,
        grid=((VOCAB + P_BLOCK - 1) // P_BLOCK,),
        in_specs=[
            pl.BlockSpec((EMBED, P_BLOCK), lambda i: (0, i)),
            pl.BlockSpec((EMBED, 128), lambda i: (0, 0)),
        ],
        out_specs=pl.BlockSpec((P_BLOCK, 128), lambda i: (i, 0)),
        out_shape=jax.ShapeDtypeStruct((VOCAB, 128), jnp.float32),
    )(embt, w1rep)

    sums = _sc_pool(tok, p)

    out = pl.pallas_call(
        _mlp_body,
        out_shape=jax.ShapeDtypeStruct((BATCH, 1), jnp.float32),
    )(sums, b1.reshape(1, HIDDEN), W2, b2.reshape(1, 1))
    return out


# speedup vs baseline: 3.1005x; 1.0138x over previous
import functools

import jax
import jax.numpy as jnp
from jax import lax
from jax.experimental import pallas as pl
from jax.experimental.pallas import tpu as pltpu
from jax.experimental.pallas import tpu_sc as plsc

VOCAB = 1000000
EMBED = 300
BATCH = 1024
SEQ = 512
HIDDEN = 16

NC = 2
NS = 16
NW = NC * NS
ROWS_PER_W = BATCH // NW
CHUNK = 128
CHUNKS_PER_ROW = SEQ // CHUNK
CHUNKS_PER_W = ROWS_PER_W * CHUNKS_PER_ROW

P_BLOCK = 12288


def _sc_pool_body(tok_hbm, p_hbm, out_hbm, tok_v, idx_v, buf_v, acc_v, sems):
    wid = lax.axis_index("s") * NC + lax.axis_index("c")
    pltpu.sync_copy(tok_hbm.at[pl.ds(wid * CHUNKS_PER_W, CHUNKS_PER_W)], tok_v)

    def copy_idx(c, parity):
        for v in range(8):
            idx_v[parity, pl.ds(16 * v, 16)] = tok_v[c, pl.ds(16 * v, 16)]

    def start_gather(parity):
        pltpu.make_async_copy(p_hbm.at[idx_v.at[parity]], buf_v.at[parity],
                              sems.at[parity]).start()

    def wait_gather(parity):
        pltpu.make_async_copy(p_hbm.at[idx_v.at[parity]], buf_v.at[parity],
                              sems.at[parity]).wait()

    copy_idx(0, 0)
    start_gather(0)

    def chunk_body(c, parity):
        @pl.when(c < CHUNKS_PER_W - 1)
        def _():
            copy_idx(c + 1, 1 - parity)
            start_gather(1 - parity)
        wait_gather(parity)
        racc = c // CHUNKS_PER_ROW

        def accum8(r, acc):
            for rr in range(8):
                acc = acc + buf_v[parity, 8 * r + rr, pl.ds(0, 16)]
            return acc

        acc = lax.fori_loop(0, CHUNK // 8, accum8,
                            jnp.zeros((16,), jnp.float32))
        acc_v[racc, pl.ds(0, 16)] = acc_v[racc, pl.ds(0, 16)] + acc

    def pair_body(g, carry):
        chunk_body(2 * g, 0)
        chunk_body(2 * g + 1, 1)
        return carry

    def zero_row(i, carry):
        acc_v[i, pl.ds(0, 16)] = jnp.zeros((16,), jnp.float32)
        return carry
    lax.fori_loop(0, ROWS_PER_W, zero_row, 0)

    lax.fori_loop(0, CHUNKS_PER_W // 2, pair_body, 0)

    pltpu.sync_copy(acc_v, out_hbm.at[pl.ds(wid * ROWS_PER_W, ROWS_PER_W)])


_sc_pool = functools.partial(
    pl.kernel,
    mesh=plsc.VectorSubcoreMesh(core_axis_name="c", subcore_axis_name="s"),
    out_type=jax.ShapeDtypeStruct((BATCH, HIDDEN), jnp.float32),
    scratch_types=[
        pltpu.VMEM((CHUNKS_PER_W, CHUNK), jnp.int32),
        pltpu.VMEM((2, CHUNK), jnp.int32),
        pltpu.VMEM((2, CHUNK, 128), jnp.float32),
        pltpu.VMEM((ROWS_PER_W, HIDDEN), jnp.float32),
        pltpu.SemaphoreType.DMA((2,)),
    ],
)(_sc_pool_body)


def _proj_body(xt_ref, w_ref, o_ref):
    o_ref[...] = lax.dot_general(
        xt_ref[...], w_ref[...], (((0,), (0,)), ((), ())),
        preferred_element_type=jnp.float32)


def _mlp_body(x_ref, b1_ref, w2_ref, b2_ref, o_ref):
    h = jnp.maximum(x_ref[...] + b1_ref[...], 0.0)
    o = jnp.dot(h, w2_ref[...], preferred_element_type=jnp.float32)
    o_ref[...] = jax.nn.sigmoid(o + b2_ref[...])


def kernel(tokens, emb_table, W1, b1, W2, b2):
    tok = tokens.reshape(BATCH * CHUNKS_PER_ROW, CHUNK)

    embt = emb_table.T
    w1rep = jnp.tile(W1 * (1.0 / SEQ), (1, 128 // HIDDEN))
    p = pl.pallas_call(
        _proj_body,
        grid=((VOCAB + P_BLOCK - 1) // P_BLOCK,),
        in_specs=[
            pl.BlockSpec((EMBED, P_BLOCK), lambda i: (0, i)),
            pl.BlockSpec((EMBED, 128), lambda i: (0, 0)),
        ],
        out_specs=pl.BlockSpec((P_BLOCK, 128), lambda i: (i, 0)),
        out_shape=jax.ShapeDtypeStruct((VOCAB, 128), jnp.float32),
    )(embt, w1rep)

    sums = _sc_pool(tok, p)

    out = pl.pallas_call(
        _mlp_body,
        out_shape=jax.ShapeDtypeStruct((BATCH, 1), jnp.float32),
    )(sums, b1.reshape(1, HIDDEN), W2, b2.reshape(1, 1))
    return out


# speedup vs baseline: 3.4339x; 1.1075x over previous
import functools

import jax
import jax.numpy as jnp
from jax import lax
from jax.experimental import pallas as pl
from jax.experimental.pallas import tpu as pltpu
from jax.experimental.pallas import tpu_sc as plsc

VOCAB = 1000000
EMBED = 300
BATCH = 1024
SEQ = 512
HIDDEN = 16

NC = 2
NS = 16
NW = NC * NS
ROWS_PER_W = BATCH // NW
CHUNK = 128
CHUNKS_PER_ROW = SEQ // CHUNK
CHUNKS_PER_W = ROWS_PER_W * CHUNKS_PER_ROW

P_BLOCK = 12288


def _sc_pool_body(tok_hbm, p_hbm, out_hbm, tok_v, idx_v, offs_v, buf_v, acc_v,
                  sems):
    wid = lax.axis_index("s") * NC + lax.axis_index("c")
    pltpu.sync_copy(tok_hbm.at[pl.ds(wid * CHUNKS_PER_W, CHUNKS_PER_W)], tok_v)

    def copy_idx(c, parity):
        for v in range(8):
            tv = tok_v[c, pl.ds(16 * v, 16)]
            idx_v[parity, pl.ds(16 * v, 16)] = tv >> 3
            offs_v[parity, pl.ds(16 * v, 16)] = (tv & 7) * 16

    def start_gather(parity):
        pltpu.make_async_copy(p_hbm.at[idx_v.at[parity]], buf_v.at[parity],
                              sems.at[parity]).start()

    def wait_gather(parity):
        pltpu.make_async_copy(p_hbm.at[idx_v.at[parity]], buf_v.at[parity],
                              sems.at[parity]).wait()

    copy_idx(0, 0)
    start_gather(0)

    def chunk_body(c, parity):
        @pl.when(c < CHUNKS_PER_W - 1)
        def _():
            copy_idx(c + 1, 1 - parity)
            start_gather(1 - parity)
        wait_gather(parity)
        racc = c // CHUNKS_PER_ROW
        iota16 = lax.iota(jnp.int32, 16)
        par = jnp.full((16,), parity, jnp.int32)

        def accum4(r, acc):
            for rr in range(4):
                sj = lax.broadcast(4 * r + rr, (16,))
                off = plsc.load_gather(offs_v, [par, sj])
                v = plsc.load_gather(buf_v, [par, sj, off + iota16])
                acc = acc + v
            return acc

        acc = lax.fori_loop(0, CHUNK // 4, accum4,
                            jnp.zeros((16,), jnp.float32))
        acc_v[racc, pl.ds(0, 16)] = acc_v[racc, pl.ds(0, 16)] + acc

    def pair_body(g, carry):
        chunk_body(2 * g, 0)
        chunk_body(2 * g + 1, 1)
        return carry

    def zero_row(i, carry):
        acc_v[i, pl.ds(0, 16)] = jnp.zeros((16,), jnp.float32)
        return carry
    lax.fori_loop(0, ROWS_PER_W, zero_row, 0)

    lax.fori_loop(0, CHUNKS_PER_W // 2, pair_body, 0)

    pltpu.sync_copy(acc_v, out_hbm.at[pl.ds(wid * ROWS_PER_W, ROWS_PER_W)])


_sc_pool = functools.partial(
    pl.kernel,
    mesh=plsc.VectorSubcoreMesh(core_axis_name="c", subcore_axis_name="s"),
    out_type=jax.ShapeDtypeStruct((BATCH, HIDDEN), jnp.float32),
    scratch_types=[
        pltpu.VMEM((CHUNKS_PER_W, CHUNK), jnp.int32),
        pltpu.VMEM((2, CHUNK), jnp.int32),
        pltpu.VMEM((2, CHUNK), jnp.int32),
        pltpu.VMEM((2, CHUNK, 128), jnp.float32),
        pltpu.VMEM((ROWS_PER_W, HIDDEN), jnp.float32),
        pltpu.SemaphoreType.DMA((2,)),
    ],
    compiler_params=pltpu.CompilerParams(needs_layout_passes=False),
)(_sc_pool_body)


def _proj_body(xt_ref, w_ref, o_ref):
    val = lax.dot_general(
        xt_ref[...], w_ref[...], (((0,), (0,)), ((), ())),
        preferred_element_type=jnp.float32)
    v3 = val.reshape(P_BLOCK // 8, 8, 128)
    lane = lax.broadcasted_iota(jnp.int32, (P_BLOCK // 8, 128), 1)
    out = jnp.zeros((P_BLOCK // 8, 128), jnp.float32)
    for a in range(8):
        va = lax.squeeze(lax.slice_in_dim(v3, a, a + 1, axis=1), (1,))
        out = jnp.where((lane >> 4) == a, va, out)
    o_ref[...] = out


def _mlp_body(x_ref, b1_ref, w2_ref, b2_ref, o_ref):
    h = jnp.maximum(x_ref[...] + b1_ref[...], 0.0)
    o = jnp.dot(h, w2_ref[...], preferred_element_type=jnp.float32)
    o_ref[...] = jax.nn.sigmoid(o + b2_ref[...])


def kernel(tokens, emb_table, W1, b1, W2, b2):
    tok = tokens.reshape(BATCH * CHUNKS_PER_ROW, CHUNK)

    embt = emb_table.T
    w1rep = jnp.tile(W1 * (1.0 / SEQ), (1, 128 // HIDDEN))
    p = pl.pallas_call(
        _proj_body,
        grid=((VOCAB + P_BLOCK - 1) // P_BLOCK,),
        in_specs=[
            pl.BlockSpec((EMBED, P_BLOCK), lambda i: (0, i)),
            pl.BlockSpec((EMBED, 128), lambda i: (0, 0)),
        ],
        out_specs=pl.BlockSpec((P_BLOCK // 8, 128), lambda i: (i, 0)),
        out_shape=jax.ShapeDtypeStruct((VOCAB // 8, 128), jnp.float32),
    )(embt, w1rep)

    sums = _sc_pool(tok, p)

    out = pl.pallas_call(
        _mlp_body,
        out_shape=jax.ShapeDtypeStruct((BATCH, 1), jnp.float32),
    )(sums, b1.reshape(1, HIDDEN), W2, b2.reshape(1, 1))
    return out


# speedup vs baseline: 3.8660x; 1.1258x over previous
import functools

import jax
import jax.numpy as jnp
from jax import lax
from jax.experimental import pallas as pl
from jax.experimental.pallas import tpu as pltpu
from jax.experimental.pallas import tpu_sc as plsc

VOCAB = 1000000
EMBED = 300
BATCH = 1024
SEQ = 512
HIDDEN = 16

NC = 2
NS = 16
NW = NC * NS
ROWS_PER_W = BATCH // NW
CHUNK = 128
CHUNKS_PER_ROW = SEQ // CHUNK
CHUNKS_PER_W = ROWS_PER_W * CHUNKS_PER_ROW

P_BLOCK = 12288


def _sc_pool_body(tok_hbm, p_hbm, out_hbm, tok_v, idx_v, buf_v, acc_v, sems):
    wid = lax.axis_index("s") * NC + lax.axis_index("c")
    pltpu.sync_copy(tok_hbm.at[pl.ds(wid * CHUNKS_PER_W, CHUNKS_PER_W)], tok_v)

    def copy_idx(c, parity):
        for v in range(8):
            idx_v[parity, pl.ds(16 * v, 16)] = tok_v[c, pl.ds(16 * v, 16)]

    def start_gather(parity):
        pltpu.make_async_copy(p_hbm.at[idx_v.at[parity]], buf_v.at[parity],
                              sems.at[parity]).start()

    def wait_gather(parity):
        pltpu.make_async_copy(p_hbm.at[idx_v.at[parity]], buf_v.at[parity],
                              sems.at[parity]).wait()

    copy_idx(0, 0)
    start_gather(0)

    def chunk_body(c, parity):
        @pl.when(c < CHUNKS_PER_W - 1)
        def _():
            copy_idx(c + 1, 1 - parity)
            start_gather(1 - parity)
        wait_gather(parity)
        racc = c // CHUNKS_PER_ROW

        def accum8(r, acc):
            for rr in range(8):
                acc = acc + buf_v[parity, 8 * r + rr, pl.ds(0, 16)]
            return acc

        acc = lax.fori_loop(0, CHUNK // 8, accum8,
                            jnp.zeros((16,), jnp.float32))
        acc_v[racc, pl.ds(0, 16)] = acc_v[racc, pl.ds(0, 16)] + acc

    def pair_body(g, carry):
        chunk_body(2 * g, 0)
        chunk_body(2 * g + 1, 1)
        return carry

    def zero_row(i, carry):
        acc_v[i, pl.ds(0, 16)] = jnp.zeros((16,), jnp.float32)
        return carry
    lax.fori_loop(0, ROWS_PER_W, zero_row, 0)

    lax.fori_loop(0, CHUNKS_PER_W // 2, pair_body, 0)

    pltpu.sync_copy(acc_v, out_hbm.at[pl.ds(wid * ROWS_PER_W, ROWS_PER_W)])


_sc_pool = functools.partial(
    pl.kernel,
    mesh=plsc.VectorSubcoreMesh(core_axis_name="c", subcore_axis_name="s"),
    out_type=jax.ShapeDtypeStruct((BATCH, HIDDEN), jnp.float32),
    scratch_types=[
        pltpu.VMEM((CHUNKS_PER_W, CHUNK), jnp.int32),
        pltpu.VMEM((2, CHUNK), jnp.int32),
        pltpu.VMEM((2, CHUNK, HIDDEN), jnp.float32),
        pltpu.VMEM((ROWS_PER_W, HIDDEN), jnp.float32),
        pltpu.SemaphoreType.DMA((2,)),
    ],
    compiler_params=pltpu.CompilerParams(use_tc_tiling_on_sc=False),
)(_sc_pool_body)


def _proj_body(xt_ref, w_ref, o_ref):
    val = lax.dot_general(
        xt_ref[...], w_ref[...], (((0,), (0,)), ((), ())),
        preferred_element_type=jnp.float32)
    v3 = val.reshape(P_BLOCK // 8, 8, 128)
    lane = lax.broadcasted_iota(jnp.int32, (P_BLOCK // 8, 128), 1)
    out = jnp.zeros((P_BLOCK // 8, 128), jnp.float32)
    for a in range(8):
        va = lax.squeeze(lax.slice_in_dim(v3, a, a + 1, axis=1), (1,))
        out = jnp.where((lane >> 4) == a, va, out)
    o_ref[...] = out


def _mlp_body(x_ref, b1_ref, w2_ref, b2_ref, o_ref):
    h = jnp.maximum(x_ref[...] + b1_ref[...], 0.0)
    o = jnp.dot(h, w2_ref[...], preferred_element_type=jnp.float32)
    o_ref[...] = jax.nn.sigmoid(o + b2_ref[...])


def kernel(tokens, emb_table, W1, b1, W2, b2):
    tok = tokens.reshape(BATCH * CHUNKS_PER_ROW, CHUNK)

    embt = emb_table.T
    w1rep = jnp.tile(W1 * (1.0 / SEQ), (1, 128 // HIDDEN))
    p = pl.pallas_call(
        _proj_body,
        grid=((VOCAB + P_BLOCK - 1) // P_BLOCK,),
        in_specs=[
            pl.BlockSpec((EMBED, P_BLOCK), lambda i: (0, i)),
            pl.BlockSpec((EMBED, 128), lambda i: (0, 0)),
        ],
        out_specs=pl.BlockSpec((P_BLOCK // 8, 128), lambda i: (i, 0)),
        out_shape=jax.ShapeDtypeStruct((VOCAB // 8, 128), jnp.float32),
    )(embt, w1rep)

    sums = _sc_pool(tok, p.reshape(VOCAB, HIDDEN))

    out = pl.pallas_call(
        _mlp_body,
        out_shape=jax.ShapeDtypeStruct((BATCH, 1), jnp.float32),
    )(sums, b1.reshape(1, HIDDEN), W2, b2.reshape(1, 1))
    return out


# speedup vs baseline: 4.1432x; 1.0717x over previous
import functools

import jax
import jax.numpy as jnp
from jax import lax
from jax.experimental import pallas as pl
from jax.experimental.pallas import tpu as pltpu
from jax.experimental.pallas import tpu_sc as plsc

VOCAB = 1000000
EMBED = 300
BATCH = 1024
SEQ = 512
HIDDEN = 16

NC = 2
NS = 16
NW = NC * NS
ROWS_PER_W = BATCH // NW
CHUNK = 128
CHUNKS_PER_ROW = SEQ // CHUNK
CHUNKS_PER_W = ROWS_PER_W * CHUNKS_PER_ROW

P_BLOCK = 12288


def _sc_pool_body(tok_hbm, p_hbm, out_hbm, tok_v, idx_v, buf_v, acc_v, sems):
    wid = lax.axis_index("s") * NC + lax.axis_index("c")
    pltpu.sync_copy(tok_hbm.at[pl.ds(wid * CHUNKS_PER_W, CHUNKS_PER_W)], tok_v)

    def copy_idx(c, parity):
        for v in range(8):
            idx_v[parity, pl.ds(16 * v, 16)] = tok_v[c, pl.ds(16 * v, 16)]

    def start_gather(parity):
        pltpu.make_async_copy(p_hbm.at[idx_v.at[parity]], buf_v.at[parity],
                              sems.at[parity]).start()

    def wait_gather(parity):
        pltpu.make_async_copy(p_hbm.at[idx_v.at[parity]], buf_v.at[parity],
                              sems.at[parity]).wait()

    copy_idx(0, 0)
    start_gather(0)

    def chunk_body(c, parity):
        @pl.when(c < CHUNKS_PER_W - 1)
        def _():
            copy_idx(c + 1, 1 - parity)
            start_gather(1 - parity)
        wait_gather(parity)
        racc = c // CHUNKS_PER_ROW

        def accum8(r, acc):
            for rr in range(8):
                acc = acc + buf_v[parity, 8 * r + rr, pl.ds(0, 16)]
            return acc

        acc = lax.fori_loop(0, CHUNK // 8, accum8,
                            jnp.zeros((16,), jnp.float32))
        acc_v[racc, pl.ds(0, 16)] = acc_v[racc, pl.ds(0, 16)] + acc

    def pair_body(g, carry):
        chunk_body(2 * g, 0)
        chunk_body(2 * g + 1, 1)
        return carry

    def zero_row(i, carry):
        acc_v[i, pl.ds(0, 16)] = jnp.zeros((16,), jnp.float32)
        return carry
    lax.fori_loop(0, ROWS_PER_W, zero_row, 0)

    lax.fori_loop(0, CHUNKS_PER_W // 2, pair_body, 0)

    pltpu.sync_copy(acc_v, out_hbm.at[pl.ds(wid * ROWS_PER_W, ROWS_PER_W)])


_sc_pool = functools.partial(
    pl.kernel,
    mesh=plsc.VectorSubcoreMesh(core_axis_name="c", subcore_axis_name="s"),
    out_type=jax.ShapeDtypeStruct((BATCH, HIDDEN), jnp.float32),
    scratch_types=[
        pltpu.VMEM((CHUNKS_PER_W, CHUNK), jnp.int32),
        pltpu.VMEM((2, CHUNK), jnp.int32),
        pltpu.VMEM((2, CHUNK, HIDDEN), jnp.float32),
        pltpu.VMEM((ROWS_PER_W, HIDDEN), jnp.float32),
        pltpu.SemaphoreType.DMA((2,)),
    ],
    compiler_params=pltpu.CompilerParams(use_tc_tiling_on_sc=False),
)(_sc_pool_body)


def _proj_body(xt_ref, w_ref, o_ref):
    val = lax.dot_general(
        xt_ref[...].astype(jnp.bfloat16), w_ref[...].astype(jnp.bfloat16),
        (((0,), (0,)), ((), ())), preferred_element_type=jnp.float32)
    v3 = val.reshape(P_BLOCK // 8, 8, 128)
    lane = lax.broadcasted_iota(jnp.int32, (P_BLOCK // 8, 128), 1)
    out = jnp.zeros((P_BLOCK // 8, 128), jnp.float32)
    for a in range(8):
        va = lax.squeeze(lax.slice_in_dim(v3, a, a + 1, axis=1), (1,))
        out = jnp.where((lane >> 4) == a, va, out)
    o_ref[...] = out


def _mlp_body(x_ref, b1_ref, w2_ref, b2_ref, o_ref):
    h = jnp.maximum(x_ref[...] + b1_ref[...], 0.0)
    o = jnp.dot(h, w2_ref[...], preferred_element_type=jnp.float32)
    o_ref[...] = jax.nn.sigmoid(o + b2_ref[...])


def kernel(tokens, emb_table, W1, b1, W2, b2):
    tok = tokens.reshape(BATCH * CHUNKS_PER_ROW, CHUNK)

    embt = emb_table.T
    w1rep = jnp.tile(W1 * (1.0 / SEQ), (1, 128 // HIDDEN))
    p = pl.pallas_call(
        _proj_body,
        grid=((VOCAB + P_BLOCK - 1) // P_BLOCK,),
        in_specs=[
            pl.BlockSpec((EMBED, P_BLOCK), lambda i: (0, i)),
            pl.BlockSpec((EMBED, 128), lambda i: (0, 0)),
        ],
        out_specs=pl.BlockSpec((P_BLOCK // 8, 128), lambda i: (i, 0)),
        out_shape=jax.ShapeDtypeStruct((VOCAB // 8, 128), jnp.float32),
    )(embt, w1rep)

    sums = _sc_pool(tok, p.reshape(VOCAB, HIDDEN))

    out = pl.pallas_call(
        _mlp_body,
        out_shape=jax.ShapeDtypeStruct((BATCH, 1), jnp.float32),
    )(sums, b1.reshape(1, HIDDEN), W2, b2.reshape(1, 1))
    return out


# speedup vs baseline: 4.1803x; 1.0090x over previous
import functools

import jax
import jax.numpy as jnp
from jax import lax
from jax.experimental import pallas as pl
from jax.experimental.pallas import tpu as pltpu
from jax.experimental.pallas import tpu_sc as plsc

VOCAB = 1000000
EMBED = 300
BATCH = 1024
SEQ = 512
HIDDEN = 16

NC = 2
NS = 16
NW = NC * NS
ROWS_PER_W = BATCH // NW
CHUNK = 128
CHUNKS_PER_ROW = SEQ // CHUNK
CHUNKS_PER_W = ROWS_PER_W * CHUNKS_PER_ROW

P_BLOCK = 16384


def _sc_pool_body(tok_hbm, p_hbm, out_hbm, tok_v, idx_v, buf_v, acc_v, sems):
    wid = lax.axis_index("s") * NC + lax.axis_index("c")
    pltpu.sync_copy(tok_hbm.at[pl.ds(wid * CHUNKS_PER_W, CHUNKS_PER_W)], tok_v)

    def copy_idx(c, parity):
        for v in range(8):
            idx_v[parity, pl.ds(16 * v, 16)] = tok_v[c, pl.ds(16 * v, 16)]

    def start_gather(parity):
        pltpu.make_async_copy(p_hbm.at[idx_v.at[parity]], buf_v.at[parity],
                              sems.at[parity]).start()

    def wait_gather(parity):
        pltpu.make_async_copy(p_hbm.at[idx_v.at[parity]], buf_v.at[parity],
                              sems.at[parity]).wait()

    copy_idx(0, 0)
    start_gather(0)

    def chunk_body(c, parity):
        @pl.when(c < CHUNKS_PER_W - 1)
        def _():
            copy_idx(c + 1, 1 - parity)
            start_gather(1 - parity)
        wait_gather(parity)
        racc = c // CHUNKS_PER_ROW

        def accum8(r, acc):
            for rr in range(8):
                acc = acc + buf_v[parity, 8 * r + rr, pl.ds(0, 16)]
            return acc

        acc = lax.fori_loop(0, CHUNK // 8, accum8,
                            jnp.zeros((16,), jnp.float32))
        acc_v[racc, pl.ds(0, 16)] = acc_v[racc, pl.ds(0, 16)] + acc

    def pair_body(g, carry):
        chunk_body(2 * g, 0)
        chunk_body(2 * g + 1, 1)
        return carry

    def zero_row(i, carry):
        acc_v[i, pl.ds(0, 16)] = jnp.zeros((16,), jnp.float32)
        return carry
    lax.fori_loop(0, ROWS_PER_W, zero_row, 0)

    lax.fori_loop(0, CHUNKS_PER_W // 2, pair_body, 0)

    pltpu.sync_copy(acc_v, out_hbm.at[pl.ds(wid * ROWS_PER_W, ROWS_PER_W)])


_sc_pool = functools.partial(
    pl.kernel,
    mesh=plsc.VectorSubcoreMesh(core_axis_name="c", subcore_axis_name="s"),
    out_type=jax.ShapeDtypeStruct((BATCH, HIDDEN), jnp.float32),
    scratch_types=[
        pltpu.VMEM((CHUNKS_PER_W, CHUNK), jnp.int32),
        pltpu.VMEM((2, CHUNK), jnp.int32),
        pltpu.VMEM((2, CHUNK, HIDDEN), jnp.float32),
        pltpu.VMEM((ROWS_PER_W, HIDDEN), jnp.float32),
        pltpu.SemaphoreType.DMA((2,)),
    ],
    compiler_params=pltpu.CompilerParams(use_tc_tiling_on_sc=False),
)(_sc_pool_body)


def _proj_body(xt_ref, w_ref, o_ref):
    val = lax.dot_general(
        xt_ref[...].astype(jnp.bfloat16), w_ref[...].astype(jnp.bfloat16),
        (((0,), (0,)), ((), ())), preferred_element_type=jnp.float32)
    v3 = val.reshape(P_BLOCK // 8, 8, 128)
    lane = lax.broadcasted_iota(jnp.int32, (P_BLOCK // 8, 128), 1)
    out = jnp.zeros((P_BLOCK // 8, 128), jnp.float32)
    for a in range(8):
        va = lax.squeeze(lax.slice_in_dim(v3, a, a + 1, axis=1), (1,))
        out = jnp.where((lane >> 4) == a, va, out)
    o_ref[...] = out


def _mlp_body(x_ref, b1_ref, w2_ref, b2_ref, o_ref):
    h = jnp.maximum(x_ref[...] + b1_ref[...], 0.0)
    o = jnp.dot(h, w2_ref[...], preferred_element_type=jnp.float32)
    o_ref[...] = jax.nn.sigmoid(o + b2_ref[...])


def kernel(tokens, emb_table, W1, b1, W2, b2):
    tok = tokens.reshape(BATCH * CHUNKS_PER_ROW, CHUNK)

    embt = emb_table.T
    w1rep = jnp.tile(W1 * (1.0 / SEQ), (1, 128 // HIDDEN))
    p = pl.pallas_call(
        _proj_body,
        grid=((VOCAB + P_BLOCK - 1) // P_BLOCK,),
        in_specs=[
            pl.BlockSpec((EMBED, P_BLOCK), lambda i: (0, i)),
            pl.BlockSpec((EMBED, 128), lambda i: (0, 0)),
        ],
        out_specs=pl.BlockSpec((P_BLOCK // 8, 128), lambda i: (i, 0)),
        out_shape=jax.ShapeDtypeStruct((VOCAB // 8, 128), jnp.float32),
    )(embt, w1rep)

    sums = _sc_pool(tok, p.reshape(VOCAB, HIDDEN))

    out = pl.pallas_call(
        _mlp_body,
        out_shape=jax.ShapeDtypeStruct((BATCH, 1), jnp.float32),
    )(sums, b1.reshape(1, HIDDEN), W2, b2.reshape(1, 1))
    return out
